# baseline (speedup 1.0000x reference)
import functools

import jax
import jax.numpy as jnp
from jax import lax
from jax.experimental import pallas as pl
from jax.experimental.pallas import tpu as pltpu
from jax.experimental.pallas import tpu_sc as plsc

N_NODES = 10000
N_EDGES = 320000
D = 128

NC = 2
NS = 16
NW = NC * NS
CHUNK = 128
CHUNKS_PER_W = 80
NHALF = CHUNKS_PER_W // 2
EDGES_PER_W = CHUNKS_PER_W * CHUNK
E_PAD = EDGES_PER_W * NW
ROWS_PER_S = 632
N_PAD = ROWS_PER_S * NS

ROW_BLOCK = 1000


def _make_segsum(with_deg):
    def body(*refs):
        if with_deg:
            (table, sd3, zeros2, zerosv, ones_h,
             psum, pdeg, accum, dega,
             sd_a, sd_b, rows_a, rows_b, ones_v, deg_v,
             semi_a, semi_b, sem_a, sem_b) = refs
        else:
            (table, sd3, zeros2,
             psum, accum,
             sd_a, sd_b, rows_a, rows_b,
             semi_a, semi_b, sem_a, sem_b) = refs

        c = lax.axis_index("c")
        s = lax.axis_index("s")
        wid = c * NS + s
        row0 = s * ROWS_PER_S

        pltpu.sync_copy(zeros2.at[pl.ds(row0, ROWS_PER_S)],
                        accum.at[pl.ds(row0, ROWS_PER_S)])
        if with_deg:
            pltpu.sync_copy(zerosv.at[pl.ds(row0, ROWS_PER_S)], deg_v)
            pltpu.sync_copy(deg_v, dega.at[pl.ds(row0, ROWS_PER_S)])
            pltpu.sync_copy(ones_h, ones_v)
        plsc.subcore_barrier()

        def start_idx(j, sd, semi):
            pltpu.make_async_copy(sd3.at[wid, j], sd, semi).start()

        def wait_idx(sd, semi):
            pltpu.make_async_copy(sd3.at[wid, 0], sd, semi).wait()

        def start_gather(sd, rows, sem):
            pltpu.make_async_copy(table.at[sd.at[0]], rows, sem).start()

        def wait_gather(rows, sem):
            pltpu.make_async_copy(table.at[pl.ds(0, CHUNK)], rows, sem).wait()

        def scatter(sd, rows):
            pltpu.sync_copy(rows, accum.at[sd.at[1]], add=True)
            if with_deg:
                pltpu.sync_copy(ones_v, dega.at[sd.at[1]], add=True)

        def process(j, sd_x, semi_x, rows_x, sem_x, sd_y, semi_y, rows_y, sem_y):
            wait_gather(rows_x, sem_x)

            @pl.when(j + 1 < CHUNKS_PER_W)
            def _():
                wait_idx(sd_y, semi_y)
                start_gather(sd_y, rows_y, sem_y)

            scatter(sd_x, rows_x)

            @pl.when(j + 2 < CHUNKS_PER_W)
            def _():
                start_idx(j + 2, sd_x, semi_x)

        start_idx(0, sd_a, semi_a)
        wait_idx(sd_a, semi_a)
        start_gather(sd_a, rows_a, sem_a)
        start_idx(1, sd_b, semi_b)

        def loop_body(i, carry):
            j = 2 * i
            process(j, sd_a, semi_a, rows_a, sem_a, sd_b, semi_b, rows_b, sem_b)
            process(j + 1, sd_b, semi_b, rows_b, sem_b, sd_a, semi_a, rows_a, sem_a)
            return carry

        lax.fori_loop(0, NHALF, loop_body, 0)
        plsc.subcore_barrier()

        pltpu.sync_copy(accum.at[pl.ds(row0, ROWS_PER_S)],
                        psum.at[c, pl.ds(row0, ROWS_PER_S)])
        if with_deg:
            pltpu.sync_copy(dega.at[pl.ds(row0, ROWS_PER_S)], deg_v)
            pltpu.sync_copy(deg_v,
                            pdeg.at[pl.ds(c * N_PAD + row0, ROWS_PER_S)])

    out_type = [jax.ShapeDtypeStruct((NC, N_PAD, D), jnp.float32)]
    scratch = [
        pltpu.VMEM_SHARED((N_PAD, D), jnp.float32),
    ]
    if with_deg:
        out_type.append(jax.ShapeDtypeStruct((NC * N_PAD,), jnp.float32))
        scratch.append(pltpu.VMEM_SHARED((N_PAD,), jnp.float32))
    scratch += [
        pltpu.VMEM((2, CHUNK), jnp.int32),
        pltpu.VMEM((2, CHUNK), jnp.int32),
        pltpu.VMEM((CHUNK, D), jnp.float32),
        pltpu.VMEM((CHUNK, D), jnp.float32),
    ]
    if with_deg:
        scratch += [
            pltpu.VMEM((CHUNK,), jnp.float32),
            pltpu.VMEM((ROWS_PER_S,), jnp.float32),
        ]
    scratch += [pltpu.SemaphoreType.DMA, pltpu.SemaphoreType.DMA,
                pltpu.SemaphoreType.DMA, pltpu.SemaphoreType.DMA]

    return pl.kernel(
        body,
        mesh=plsc.VectorSubcoreMesh(core_axis_name="c", subcore_axis_name="s"),
        out_type=tuple(out_type) if with_deg else out_type[0],
        scratch_types=scratch,
    )


_segsum_deg = _make_segsum(True)
_segsum_nodeg = _make_segsum(False)


def _dense1_body(p0, p1, d0, d1, x, WlT, WrT, b, out):
    deg = jnp.maximum(d0[...] + d1[...], 1.0)
    agg = (p0[...] + p1[...]) / deg
    h = (jnp.dot(agg, WlT[...], preferred_element_type=jnp.float32)
         + jnp.dot(x[...], WrT[...], preferred_element_type=jnp.float32)
         + b[...])
    out[...] = jnp.maximum(h, 0.0)


def _dense2_body(p0, p1, d0, d1, x, WlT, WrT, b, WoT, bo, out):
    deg = jnp.maximum(d0[...] + d1[...], 1.0)
    agg = (p0[...] + p1[...]) / deg
    h = (jnp.dot(agg, WlT[...], preferred_element_type=jnp.float32)
         + jnp.dot(x[...], WrT[...], preferred_element_type=jnp.float32)
         + b[...])
    h = jnp.maximum(h, 0.0)
    out[...] = jnp.dot(h, WoT[...], preferred_element_type=jnp.float32) + bo[...]


def _row_specs():
    blk = lambda i: (i, 0)
    full = lambda i: (0, 0)
    return [
        pl.BlockSpec((ROW_BLOCK, D), blk),
        pl.BlockSpec((ROW_BLOCK, D), blk),
        pl.BlockSpec((ROW_BLOCK, 1), blk),
        pl.BlockSpec((ROW_BLOCK, 1), blk),
        pl.BlockSpec((ROW_BLOCK, D), blk),
        pl.BlockSpec((D, D), full),
        pl.BlockSpec((D, D), full),
        pl.BlockSpec((1, D), full),
    ]


def _dense1(p0, p1, d0, d1, x, WlT, WrT, b):
    grid = N_NODES // ROW_BLOCK
    return pl.pallas_call(
        _dense1_body,
        grid=(grid,),
        in_specs=_row_specs(),
        out_specs=pl.BlockSpec((ROW_BLOCK, D), lambda i: (i, 0)),
        out_shape=jax.ShapeDtypeStruct((N_NODES, D), jnp.float32),
    )(p0, p1, d0, d1, x, WlT, WrT, b)


def _dense2(p0, p1, d0, d1, x, WlT, WrT, b, WoT, bo):
    grid = N_NODES // ROW_BLOCK
    n_out = WoT.shape[1]
    in_specs = _row_specs() + [
        pl.BlockSpec((D, n_out), lambda i: (0, 0)),
        pl.BlockSpec((1, n_out), lambda i: (0, 0)),
    ]
    return pl.pallas_call(
        _dense2_body,
        grid=(grid,),
        in_specs=in_specs,
        out_specs=pl.BlockSpec((ROW_BLOCK, n_out), lambda i: (i, 0)),
        out_shape=jax.ShapeDtypeStruct((N_NODES, n_out), jnp.float32),
    )(p0, p1, d0, d1, x, WlT, WrT, b, WoT, bo)


def kernel(x, edge_index, W1l, b1, W1r, W2l, b2, W2r, Wlin, blin):
    ei = edge_index.astype(jnp.int32)
    pad = E_PAD - N_EDGES
    src = jnp.concatenate([ei[0], jnp.zeros((pad,), jnp.int32)])
    dst = jnp.concatenate([ei[1], jnp.full((pad,), N_NODES, jnp.int32)])
    src3 = src.reshape(NW, CHUNKS_PER_W, 1, CHUNK)
    dst3 = dst.reshape(NW, CHUNKS_PER_W, 1, CHUNK)
    sd3 = jnp.concatenate([src3, dst3], axis=2)
    zeros2 = jnp.zeros((N_PAD, D), jnp.float32)
    zerosv = jnp.zeros((N_PAD,), jnp.float32)
    ones_h = jnp.ones((CHUNK,), jnp.float32)

    psum1, pdeg = _segsum_deg(x, sd3, zeros2, zerosv, ones_h)
    pdeg = pdeg.reshape(NC, N_PAD)
    d0 = pdeg[0][:, None]
    d1 = pdeg[1][:, None]
    h1 = _dense1(psum1[0], psum1[1], d0, d1, x,
                 W1l.T, W1r.T, b1[None, :])

    psum2 = _segsum_nodeg(h1, sd3, zeros2)
    out = _dense2(psum2[0], psum2[1], d0, d1, h1,
                  W2l.T, W2r.T, b2[None, :], Wlin.T, blin[None, :])
    return out


# speedup vs baseline: 1.0451x; 1.0451x over previous
import functools

import jax
import jax.numpy as jnp
from jax import lax
from jax.experimental import pallas as pl
from jax.experimental.pallas import tpu as pltpu
from jax.experimental.pallas import tpu_sc as plsc

N_NODES = 10000
N_EDGES = 320000
D = 128

NC = 2
NS = 16
NW = NC * NS
CHUNK = 128
CHUNKS_PER_W = 80
NHALF = CHUNKS_PER_W // 2
EDGES_PER_W = CHUNKS_PER_W * CHUNK
E_PAD = EDGES_PER_W * NW
ROWS_PER_S = 632
N_PAD = ROWS_PER_S * NS

ROW_BLOCK = 1000


def _make_segsum(with_deg):
    def body(*refs):
        if with_deg:
            (table, sd3, zeros2, zerosv, ones_h,
             psum, pdeg, accum, dega,
             sd_v, si_a, di_a, si_b, di_b, rows_a, rows_b, ones_v, deg_v,
             sem_a, sem_b) = refs
        else:
            (table, sd3, zeros2,
             psum, accum,
             sd_v, si_a, di_a, si_b, di_b, rows_a, rows_b,
             sem_a, sem_b) = refs

        c = lax.axis_index("c")
        s = lax.axis_index("s")
        wid = c * NS + s
        row0 = s * ROWS_PER_S

        pltpu.sync_copy(zeros2.at[pl.ds(row0, ROWS_PER_S)],
                        accum.at[pl.ds(row0, ROWS_PER_S)])
        pltpu.sync_copy(sd3.at[wid], sd_v)
        if with_deg:
            pltpu.sync_copy(zerosv.at[pl.ds(row0, ROWS_PER_S)], deg_v)
            pltpu.sync_copy(deg_v, dega.at[pl.ds(row0, ROWS_PER_S)])
            pltpu.sync_copy(ones_h, ones_v)
        plsc.subcore_barrier()

        def unpack(j, si, di):
            for k in range(CHUNK // 16):
                w = sd_v[j, pl.ds(k * 16, 16)]
                si[0, pl.ds(k * 16, 16)] = w & 0xFFFF
                di[0, pl.ds(k * 16, 16)] = lax.shift_right_logical(w, 16)

        def start(si, rows, sem):
            pltpu.make_async_copy(table.at[si.at[0]], rows, sem).start()

        def wait(rows, sem):
            pltpu.make_async_copy(table.at[pl.ds(0, CHUNK)], rows, sem).wait()

        def scatter(di, rows):
            pltpu.sync_copy(rows, accum.at[di.at[0]], add=True)
            if with_deg:
                pltpu.sync_copy(ones_v, dega.at[di.at[0]], add=True)

        unpack(0, si_a, di_a)
        start(si_a, rows_a, sem_a)

        def loop_body(i, carry):
            j = 2 * i
            unpack(j + 1, si_b, di_b)
            start(si_b, rows_b, sem_b)
            wait(rows_a, sem_a)
            scatter(di_a, rows_a)
            unpack(lax.rem(j + 2, CHUNKS_PER_W), si_a, di_a)
            start(si_a, rows_a, sem_a)
            wait(rows_b, sem_b)
            scatter(di_b, rows_b)
            return carry

        lax.fori_loop(0, NHALF, loop_body, 0)
        wait(rows_a, sem_a)
        plsc.subcore_barrier()

        pltpu.sync_copy(accum.at[pl.ds(row0, ROWS_PER_S)],
                        psum.at[c, pl.ds(row0, ROWS_PER_S)])
        if with_deg:
            pltpu.sync_copy(dega.at[pl.ds(row0, ROWS_PER_S)], deg_v)
            pltpu.sync_copy(deg_v,
                            pdeg.at[pl.ds(c * N_PAD + row0, ROWS_PER_S)])

    out_type = [jax.ShapeDtypeStruct((NC, N_PAD, D), jnp.float32)]
    scratch = [
        pltpu.VMEM_SHARED((N_PAD, D), jnp.float32),
    ]
    if with_deg:
        out_type.append(jax.ShapeDtypeStruct((NC * N_PAD,), jnp.float32))
        scratch.append(pltpu.VMEM_SHARED((N_PAD,), jnp.float32))
    scratch += [
        pltpu.VMEM((CHUNKS_PER_W, CHUNK), jnp.int32),
        pltpu.VMEM((1, CHUNK), jnp.int32),
        pltpu.VMEM((1, CHUNK), jnp.int32),
        pltpu.VMEM((1, CHUNK), jnp.int32),
        pltpu.VMEM((1, CHUNK), jnp.int32),
        pltpu.VMEM((CHUNK, D), jnp.float32),
        pltpu.VMEM((CHUNK, D), jnp.float32),
    ]
    if with_deg:
        scratch += [
            pltpu.VMEM((CHUNK,), jnp.float32),
            pltpu.VMEM((ROWS_PER_S,), jnp.float32),
        ]
    scratch += [pltpu.SemaphoreType.DMA, pltpu.SemaphoreType.DMA]

    return pl.kernel(
        body,
        mesh=plsc.VectorSubcoreMesh(core_axis_name="c", subcore_axis_name="s"),
        out_type=tuple(out_type) if with_deg else out_type[0],
        scratch_types=scratch,
    )


_segsum_deg = _make_segsum(True)
_segsum_nodeg = _make_segsum(False)


def _dense1_body(p0, p1, d0, d1, x, WlT, WrT, b, out):
    deg = jnp.maximum(d0[...] + d1[...], 1.0)
    agg = (p0[...] + p1[...]) / deg
    h = (jnp.dot(agg, WlT[...], preferred_element_type=jnp.float32)
         + jnp.dot(x[...], WrT[...], preferred_element_type=jnp.float32)
         + b[...])
    out[...] = jnp.maximum(h, 0.0)


def _dense2_body(p0, p1, d0, d1, x, WlT, WrT, b, WoT, bo, out):
    deg = jnp.maximum(d0[...] + d1[...], 1.0)
    agg = (p0[...] + p1[...]) / deg
    h = (jnp.dot(agg, WlT[...], preferred_element_type=jnp.float32)
         + jnp.dot(x[...], WrT[...], preferred_element_type=jnp.float32)
         + b[...])
    h = jnp.maximum(h, 0.0)
    out[...] = jnp.dot(h, WoT[...], preferred_element_type=jnp.float32) + bo[...]


def _row_specs():
    blk = lambda i: (i, 0)
    full = lambda i: (0, 0)
    return [
        pl.BlockSpec((ROW_BLOCK, D), blk),
        pl.BlockSpec((ROW_BLOCK, D), blk),
        pl.BlockSpec((ROW_BLOCK, 1), blk),
        pl.BlockSpec((ROW_BLOCK, 1), blk),
        pl.BlockSpec((ROW_BLOCK, D), blk),
        pl.BlockSpec((D, D), full),
        pl.BlockSpec((D, D), full),
        pl.BlockSpec((1, D), full),
    ]


def _dense1(p0, p1, d0, d1, x, WlT, WrT, b):
    grid = N_NODES // ROW_BLOCK
    return pl.pallas_call(
        _dense1_body,
        grid=(grid,),
        in_specs=_row_specs(),
        out_specs=pl.BlockSpec((ROW_BLOCK, D), lambda i: (i, 0)),
        out_shape=jax.ShapeDtypeStruct((N_NODES, D), jnp.float32),
    )(p0, p1, d0, d1, x, WlT, WrT, b)


def _dense2(p0, p1, d0, d1, x, WlT, WrT, b, WoT, bo):
    grid = N_NODES // ROW_BLOCK
    n_out = WoT.shape[1]
    in_specs = _row_specs() + [
        pl.BlockSpec((D, n_out), lambda i: (0, 0)),
        pl.BlockSpec((1, n_out), lambda i: (0, 0)),
    ]
    return pl.pallas_call(
        _dense2_body,
        grid=(grid,),
        in_specs=in_specs,
        out_specs=pl.BlockSpec((ROW_BLOCK, n_out), lambda i: (i, 0)),
        out_shape=jax.ShapeDtypeStruct((N_NODES, n_out), jnp.float32),
    )(p0, p1, d0, d1, x, WlT, WrT, b, WoT, bo)


def kernel(x, edge_index, W1l, b1, W1r, W2l, b2, W2r, Wlin, blin):
    ei = edge_index.astype(jnp.int32)
    pad = E_PAD - N_EDGES
    src = jnp.concatenate([ei[0], jnp.zeros((pad,), jnp.int32)])
    dst = jnp.concatenate([ei[1], jnp.full((pad,), N_NODES, jnp.int32)])
    sd3 = (src | (dst << 16)).reshape(NW, CHUNKS_PER_W, CHUNK)
    zeros2 = jnp.zeros((N_PAD, D), jnp.float32)
    zerosv = jnp.zeros((N_PAD,), jnp.float32)
    ones_h = jnp.ones((CHUNK,), jnp.float32)

    psum1, pdeg = _segsum_deg(x, sd3, zeros2, zerosv, ones_h)
    pdeg = pdeg.reshape(NC, N_PAD)
    d0 = pdeg[0][:, None]
    d1 = pdeg[1][:, None]
    h1 = _dense1(psum1[0], psum1[1], d0, d1, x,
                 W1l.T, W1r.T, b1[None, :])

    psum2 = _segsum_nodeg(h1, sd3, zeros2)
    out = _dense2(psum2[0], psum2[1], d0, d1, h1,
                  W2l.T, W2r.T, b2[None, :], Wlin.T, blin[None, :])
    return out


# speedup vs baseline: 1.2522x; 1.1982x over previous
import jax
import jax.numpy as jnp
from jax import lax
from jax.experimental import pallas as pl
from jax.experimental.pallas import tpu as pltpu
from jax.experimental.pallas import tpu_sc as plsc

N_NODES = 10000
N_EDGES = 320000
D = 128
DH = D // 2

NC = 2
NS = 16
CHUNK = 128
KDEPTH = 4
CHUNKS_PER_S = 160
NSUP = CHUNKS_PER_S // KDEPTH
EDGES_PER_S = CHUNKS_PER_S * CHUNK
E_PAD = EDGES_PER_S * NS
ROWS_PER_S = 632
N_PAD = ROWS_PER_S * NS

ROW_BLOCK = 1000


def _make_segsum(with_deg):
    def body(*refs):
        if with_deg:
            (table, src3, dst3, zeros2, zerosv, ones_h,
             psum, pdeg, accum, dega,
             src_v, dst_v, ones_v, deg_v) = refs[:14]
            bufs = refs[14:14 + KDEPTH]
            sem_g, sem_s, sem_d = refs[14 + KDEPTH:]
        else:
            (table, src3, dst3, zeros2,
             psum, accum,
             src_v, dst_v) = refs[:8]
            bufs = refs[8:8 + KDEPTH]
            sem_g, sem_s = refs[8 + KDEPTH:]

        c = lax.axis_index("c")
        s = lax.axis_index("s")
        row0 = s * ROWS_PER_S

        pltpu.sync_copy(zeros2.at[pl.ds(row0, ROWS_PER_S)],
                        accum.at[pl.ds(row0, ROWS_PER_S)])
        pltpu.sync_copy(src3.at[c, s], src_v)
        pltpu.sync_copy(dst3.at[s], dst_v)
        if with_deg:
            pltpu.sync_copy(zerosv.at[pl.ds(row0, ROWS_PER_S)], deg_v)
            pltpu.sync_copy(deg_v, dega.at[pl.ds(row0, ROWS_PER_S)])
            pltpu.sync_copy(ones_h, ones_v)
        plsc.subcore_barrier()

        def drain(buf, sem):
            pltpu.make_async_copy(table.at[pl.ds(0, CHUNK)], buf, sem).wait()

        def sup_body(i, carry):
            base = i * KDEPTH
            for t in range(KDEPTH):
                pltpu.async_copy(table.at[src_v.at[base + t]], bufs[t], sem_g)
            for t in range(KDEPTH):
                drain(bufs[t], sem_g)
            for t in range(KDEPTH):
                pltpu.async_copy(bufs[t], accum.at[dst_v.at[base + t]],
                                 sem_s, add=True)
            if with_deg:
                @pl.when(c == 0)
                def _():
                    for t in range(KDEPTH):
                        pltpu.async_copy(ones_v, dega.at[dst_v.at[base + t]],
                                         sem_d, add=True)
            for t in range(KDEPTH):
                pltpu.make_async_copy(bufs[t], accum.at[dst_v.at[0]],
                                      sem_s).wait()
            if with_deg:
                @pl.when(c == 0)
                def _():
                    for t in range(KDEPTH):
                        pltpu.make_async_copy(ones_v, dega.at[dst_v.at[0]],
                                              sem_d).wait()
            return carry

        lax.fori_loop(0, NSUP, sup_body, 0)
        plsc.subcore_barrier()

        pltpu.sync_copy(accum.at[pl.ds(row0, ROWS_PER_S)],
                        psum.at[c, pl.ds(row0, ROWS_PER_S)])
        if with_deg:
            @pl.when(c == 0)
            def _():
                pltpu.sync_copy(dega.at[pl.ds(row0, ROWS_PER_S)], deg_v)
                pltpu.sync_copy(deg_v, pdeg.at[pl.ds(row0, ROWS_PER_S)])

    out_type = [jax.ShapeDtypeStruct((NC, N_PAD, DH), jnp.float32)]
    scratch = [
        pltpu.VMEM_SHARED((N_PAD, DH), jnp.float32),
    ]
    if with_deg:
        out_type.append(jax.ShapeDtypeStruct((N_PAD,), jnp.float32))
        scratch.append(pltpu.VMEM_SHARED((N_PAD,), jnp.float32))
    scratch += [
        pltpu.VMEM((CHUNKS_PER_S, CHUNK), jnp.int32),
        pltpu.VMEM((CHUNKS_PER_S, CHUNK), jnp.int32),
    ]
    if with_deg:
        scratch += [
            pltpu.VMEM((CHUNK,), jnp.float32),
            pltpu.VMEM((ROWS_PER_S,), jnp.float32),
        ]
    scratch += [pltpu.VMEM((CHUNK, DH), jnp.float32) for _ in range(KDEPTH)]
    scratch += [pltpu.SemaphoreType.DMA, pltpu.SemaphoreType.DMA]
    if with_deg:
        scratch.append(pltpu.SemaphoreType.DMA)

    return pl.kernel(
        body,
        mesh=plsc.VectorSubcoreMesh(core_axis_name="c", subcore_axis_name="s"),
        out_type=tuple(out_type) if with_deg else out_type[0],
        scratch_types=scratch,
        compiler_params=pltpu.CompilerParams(use_tc_tiling_on_sc=False),
    )


_segsum_deg = _make_segsum(True)
_segsum_nodeg = _make_segsum(False)


def _dense1_body(pl_, ph_, d_, x, WlTl, WlTh, WrT, b, out):
    deg = jnp.maximum(d_[...], 1.0)
    h = (jnp.dot(pl_[...] / deg, WlTl[...], preferred_element_type=jnp.float32)
         + jnp.dot(ph_[...] / deg, WlTh[...], preferred_element_type=jnp.float32)
         + jnp.dot(x[...], WrT[...], preferred_element_type=jnp.float32)
         + b[...])
    out[...] = jnp.maximum(h, 0.0)


def _dense2_body(pl_, ph_, d_, x, WlTl, WlTh, WrT, b, WoT, bo, out):
    deg = jnp.maximum(d_[...], 1.0)
    h = (jnp.dot(pl_[...] / deg, WlTl[...], preferred_element_type=jnp.float32)
         + jnp.dot(ph_[...] / deg, WlTh[...], preferred_element_type=jnp.float32)
         + jnp.dot(x[...], WrT[...], preferred_element_type=jnp.float32)
         + b[...])
    h = jnp.maximum(h, 0.0)
    out[...] = jnp.dot(h, WoT[...], preferred_element_type=jnp.float32) + bo[...]


def _row_specs():
    blk = lambda i: (i, 0)
    full = lambda i: (0, 0)
    return [
        pl.BlockSpec((ROW_BLOCK, DH), blk),
        pl.BlockSpec((ROW_BLOCK, DH), blk),
        pl.BlockSpec((ROW_BLOCK, 1), blk),
        pl.BlockSpec((ROW_BLOCK, D), blk),
        pl.BlockSpec((DH, D), full),
        pl.BlockSpec((DH, D), full),
        pl.BlockSpec((D, D), full),
        pl.BlockSpec((1, D), full),
    ]


def _dense1(pl_, ph_, d_, x, WlTl, WlTh, WrT, b):
    grid = N_NODES // ROW_BLOCK
    return pl.pallas_call(
        _dense1_body,
        grid=(grid,),
        in_specs=_row_specs(),
        out_specs=pl.BlockSpec((ROW_BLOCK, D), lambda i: (i, 0)),
        out_shape=jax.ShapeDtypeStruct((N_NODES, D), jnp.float32),
    )(pl_, ph_, d_, x, WlTl, WlTh, WrT, b)


def _dense2(pl_, ph_, d_, x, WlTl, WlTh, WrT, b, WoT, bo):
    grid = N_NODES // ROW_BLOCK
    n_out = WoT.shape[1]
    in_specs = _row_specs() + [
        pl.BlockSpec((D, n_out), lambda i: (0, 0)),
        pl.BlockSpec((1, n_out), lambda i: (0, 0)),
    ]
    return pl.pallas_call(
        _dense2_body,
        grid=(grid,),
        in_specs=in_specs,
        out_specs=pl.BlockSpec((ROW_BLOCK, n_out), lambda i: (i, 0)),
        out_shape=jax.ShapeDtypeStruct((N_NODES, n_out), jnp.float32),
    )(pl_, ph_, d_, x, WlTl, WlTh, WrT, b, WoT, bo)


def _split_table(t):
    return jnp.concatenate([t[:, :DH], t[:, DH:]], axis=0)


def kernel(x, edge_index, W1l, b1, W1r, W2l, b2, W2r, Wlin, blin):
    ei = edge_index.astype(jnp.int32)
    pad = E_PAD - N_EDGES
    src = jnp.concatenate([ei[0], jnp.zeros((pad,), jnp.int32)])
    dst = jnp.concatenate([ei[1], jnp.full((pad,), N_NODES, jnp.int32)])
    src_t = src.reshape(NS, CHUNKS_PER_S, CHUNK)
    src3 = jnp.stack([src_t, src_t + N_NODES])
    dst3 = dst.reshape(NS, CHUNKS_PER_S, CHUNK)
    zeros2 = jnp.zeros((N_PAD, DH), jnp.float32)
    zerosv = jnp.zeros((N_PAD,), jnp.float32)
    ones_h = jnp.ones((CHUNK,), jnp.float32)

    psum1, pdeg = _segsum_deg(_split_table(x), src3, dst3,
                              zeros2, zerosv, ones_h)
    d_ = pdeg[:, None]
    W1lT = W1l.T
    h1 = _dense1(psum1[0], psum1[1], d_, x,
                 W1lT[:DH], W1lT[DH:], W1r.T, b1[None, :])

    psum2 = _segsum_nodeg(_split_table(h1), src3, dst3, zeros2)
    W2lT = W2l.T
    out = _dense2(psum2[0], psum2[1], d_, h1,
                  W2lT[:DH], W2lT[DH:], W2r.T, b2[None, :],
                  Wlin.T, blin[None, :])
    return out


# speedup vs baseline: 1.6964x; 1.3548x over previous
import jax
import jax.numpy as jnp
from jax import lax
from jax.experimental import pallas as pl
from jax.experimental.pallas import tpu as pltpu
from jax.experimental.pallas import tpu_sc as plsc

N_NODES = 10000
N_EDGES = 320000
D = 128

NC = 2
NS = 16
NW = NC * NS
CHUNK = 128
KDEPTH = 5
CHUNKS_PER_W = 80
NSUP = CHUNKS_PER_W // KDEPTH
EDGES_PER_W = CHUNKS_PER_W * CHUNK
E_PAD = EDGES_PER_W * NW
ROWS_PER_S = 632
N_PAD = ROWS_PER_S * NS

ROW_BLOCK = 1000


def _make_segsum(with_deg):
    def body(*refs):
        if with_deg:
            (table, src3, dst3, zeros2, zerosv, ones_h,
             psum, pdeg, accum, dega,
             src_v, dst_v, ones_v, deg_v) = refs[:14]
            bufs = refs[14:14 + KDEPTH]
            sem_g, sem_s, sem_d = refs[14 + KDEPTH:]
        else:
            (table, src3, dst3, zeros2,
             psum, accum,
             src_v, dst_v) = refs[:8]
            bufs = refs[8:8 + KDEPTH]
            sem_g, sem_s = refs[8 + KDEPTH:]

        c = lax.axis_index("c")
        s = lax.axis_index("s")
        wid = c * NS + s
        row0 = s * ROWS_PER_S

        pltpu.sync_copy(zeros2.at[pl.ds(row0, ROWS_PER_S)],
                        accum.at[pl.ds(row0, ROWS_PER_S)])
        pltpu.sync_copy(src3.at[wid], src_v)
        pltpu.sync_copy(dst3.at[wid], dst_v)
        if with_deg:
            pltpu.sync_copy(zerosv.at[pl.ds(row0, ROWS_PER_S)], deg_v)
            pltpu.sync_copy(deg_v, dega.at[pl.ds(row0, ROWS_PER_S)])
            pltpu.sync_copy(ones_h, ones_v)
        plsc.subcore_barrier()

        def sup_body(i, carry):
            base = i * KDEPTH
            for t in range(KDEPTH):
                pltpu.async_copy(table.at[src_v.at[base + t]], bufs[t], sem_g)
            for t in range(KDEPTH):
                pltpu.make_async_copy(table.at[pl.ds(0, CHUNK)],
                                      bufs[t], sem_g).wait()
            for t in range(KDEPTH):
                pltpu.async_copy(bufs[t], accum.at[dst_v.at[base + t]],
                                 sem_s, add=True)
            if with_deg:
                for t in range(KDEPTH):
                    pltpu.async_copy(ones_v, dega.at[dst_v.at[base + t]],
                                     sem_d, add=True)
            for t in range(KDEPTH):
                pltpu.make_async_copy(bufs[t], accum.at[dst_v.at[0]],
                                      sem_s).wait()
            if with_deg:
                for t in range(KDEPTH):
                    pltpu.make_async_copy(ones_v, dega.at[dst_v.at[0]],
                                          sem_d).wait()
            return carry

        lax.fori_loop(0, NSUP, sup_body, 0)
        plsc.subcore_barrier()

        pltpu.sync_copy(accum.at[pl.ds(row0, ROWS_PER_S)],
                        psum.at[c, pl.ds(row0, ROWS_PER_S)])
        if with_deg:
            pltpu.sync_copy(dega.at[pl.ds(row0, ROWS_PER_S)], deg_v)
            pltpu.sync_copy(deg_v,
                            pdeg.at[pl.ds(c * N_PAD + row0, ROWS_PER_S)])

    out_type = [jax.ShapeDtypeStruct((NC, N_PAD, D), jnp.bfloat16)]
    scratch = [
        pltpu.VMEM_SHARED((N_PAD, D), jnp.bfloat16),
    ]
    if with_deg:
        out_type.append(jax.ShapeDtypeStruct((NC * N_PAD,), jnp.float32))
        scratch.append(pltpu.VMEM_SHARED((N_PAD,), jnp.float32))
    scratch += [
        pltpu.VMEM((CHUNKS_PER_W, CHUNK), jnp.int32),
        pltpu.VMEM((CHUNKS_PER_W, CHUNK), jnp.int32),
    ]
    if with_deg:
        scratch += [
            pltpu.VMEM((CHUNK,), jnp.float32),
            pltpu.VMEM((ROWS_PER_S,), jnp.float32),
        ]
    scratch += [pltpu.VMEM((CHUNK, D), jnp.bfloat16) for _ in range(KDEPTH)]
    scratch += [pltpu.SemaphoreType.DMA, pltpu.SemaphoreType.DMA]
    if with_deg:
        scratch.append(pltpu.SemaphoreType.DMA)

    return pl.kernel(
        body,
        mesh=plsc.VectorSubcoreMesh(core_axis_name="c", subcore_axis_name="s"),
        out_type=tuple(out_type) if with_deg else out_type[0],
        scratch_types=scratch,
        compiler_params=pltpu.CompilerParams(use_tc_tiling_on_sc=False),
    )


_segsum_deg = _make_segsum(True)
_segsum_nodeg = _make_segsum(False)


def _dense1_body(p0, p1, d0, d1, x, WlT, WrT, b, out):
    deg = jnp.maximum(d0[...] + d1[...], 1.0)
    agg = (p0[...].astype(jnp.float32) + p1[...].astype(jnp.float32)) / deg
    h = (jnp.dot(agg, WlT[...], preferred_element_type=jnp.float32)
         + jnp.dot(x[...], WrT[...], preferred_element_type=jnp.float32)
         + b[...])
    out[...] = jnp.maximum(h, 0.0)


def _dense2_body(p0, p1, d0, d1, x, WlT, WrT, b, WoT, bo, out):
    deg = jnp.maximum(d0[...] + d1[...], 1.0)
    agg = (p0[...].astype(jnp.float32) + p1[...].astype(jnp.float32)) / deg
    h = (jnp.dot(agg, WlT[...], preferred_element_type=jnp.float32)
         + jnp.dot(x[...], WrT[...], preferred_element_type=jnp.float32)
         + b[...])
    h = jnp.maximum(h, 0.0)
    out[...] = jnp.dot(h, WoT[...], preferred_element_type=jnp.float32) + bo[...]


def _row_specs():
    blk = lambda i: (i, 0)
    full = lambda i: (0, 0)
    return [
        pl.BlockSpec((ROW_BLOCK, D), blk),
        pl.BlockSpec((ROW_BLOCK, D), blk),
        pl.BlockSpec((ROW_BLOCK, 1), blk),
        pl.BlockSpec((ROW_BLOCK, 1), blk),
        pl.BlockSpec((ROW_BLOCK, D), blk),
        pl.BlockSpec((D, D), full),
        pl.BlockSpec((D, D), full),
        pl.BlockSpec((1, D), full),
    ]


def _dense1(p0, p1, d0, d1, x, WlT, WrT, b):
    grid = N_NODES // ROW_BLOCK
    return pl.pallas_call(
        _dense1_body,
        grid=(grid,),
        in_specs=_row_specs(),
        out_specs=pl.BlockSpec((ROW_BLOCK, D), lambda i: (i, 0)),
        out_shape=jax.ShapeDtypeStruct((N_NODES, D), jnp.float32),
    )(p0, p1, d0, d1, x, WlT, WrT, b)


def _dense2(p0, p1, d0, d1, x, WlT, WrT, b, WoT, bo):
    grid = N_NODES // ROW_BLOCK
    n_out = WoT.shape[1]
    in_specs = _row_specs() + [
        pl.BlockSpec((D, n_out), lambda i: (0, 0)),
        pl.BlockSpec((1, n_out), lambda i: (0, 0)),
    ]
    return pl.pallas_call(
        _dense2_body,
        grid=(grid,),
        in_specs=in_specs,
        out_specs=pl.BlockSpec((ROW_BLOCK, n_out), lambda i: (i, 0)),
        out_shape=jax.ShapeDtypeStruct((N_NODES, n_out), jnp.float32),
    )(p0, p1, d0, d1, x, WlT, WrT, b, WoT, bo)


def kernel(x, edge_index, W1l, b1, W1r, W2l, b2, W2r, Wlin, blin):
    ei = edge_index.astype(jnp.int32)
    pad = E_PAD - N_EDGES
    src = jnp.concatenate([ei[0], jnp.zeros((pad,), jnp.int32)])
    dst = jnp.concatenate([ei[1], jnp.full((pad,), N_NODES, jnp.int32)])
    src3 = src.reshape(NW, CHUNKS_PER_W, CHUNK)
    dst3 = dst.reshape(NW, CHUNKS_PER_W, CHUNK)
    zeros2 = jnp.zeros((N_PAD, D), jnp.bfloat16)
    zerosv = jnp.zeros((N_PAD,), jnp.float32)
    ones_h = jnp.ones((CHUNK,), jnp.float32)

    psum1, pdeg = _segsum_deg(x.astype(jnp.bfloat16), src3, dst3,
                              zeros2, zerosv, ones_h)
    pdeg = pdeg.reshape(NC, N_PAD)
    d0 = pdeg[0][:, None]
    d1 = pdeg[1][:, None]
    h1 = _dense1(psum1[0], psum1[1], d0, d1, x,
                 W1l.T, W1r.T, b1[None, :])

    psum2 = _segsum_nodeg(h1.astype(jnp.bfloat16), src3, dst3, zeros2)
    out = _dense2(psum2[0], psum2[1], d0, d1, h1,
                  W2l.T, W2r.T, b2[None, :], Wlin.T, blin[None, :])
    return out


# speedup vs baseline: 1.7085x; 1.0071x over previous
import jax
import jax.numpy as jnp
from jax import lax
from jax.experimental import pallas as pl
from jax.experimental.pallas import tpu as pltpu
from jax.experimental.pallas import tpu_sc as plsc

N_NODES = 10000
N_EDGES = 320000
D = 128

NC = 2
NS = 16
NW = NC * NS
CHUNK = 128
KDEPTH = 8
CHUNKS_PER_W = 80
NSUP = CHUNKS_PER_W // KDEPTH
EDGES_PER_W = CHUNKS_PER_W * CHUNK
E_PAD = EDGES_PER_W * NW
ROWS_PER_S = 632
N_PAD = ROWS_PER_S * NS

ROW_BLOCK = 1000


def _make_segsum(with_deg):
    def body(*refs):
        if with_deg:
            (table, src3, dst3, zeros2, zerosv, ones_h,
             psum, pdeg, accum, dega,
             src_v, dst_v, ones_v, deg_v) = refs[:14]
            bufs = refs[14:14 + KDEPTH]
            sem_g, sem_s, sem_d = refs[14 + KDEPTH:]
        else:
            (table, src3, dst3, zeros2,
             psum, accum,
             src_v, dst_v) = refs[:8]
            bufs = refs[8:8 + KDEPTH]
            sem_g, sem_s = refs[8 + KDEPTH:]

        c = lax.axis_index("c")
        s = lax.axis_index("s")
        wid = c * NS + s
        row0 = s * ROWS_PER_S

        pltpu.sync_copy(zeros2.at[pl.ds(row0, ROWS_PER_S)],
                        accum.at[pl.ds(row0, ROWS_PER_S)])
        pltpu.sync_copy(src3.at[wid], src_v)
        pltpu.sync_copy(dst3.at[wid], dst_v)
        if with_deg:
            pltpu.sync_copy(zerosv.at[pl.ds(row0, ROWS_PER_S)], deg_v)
            pltpu.sync_copy(deg_v, dega.at[pl.ds(row0, ROWS_PER_S)])
            pltpu.sync_copy(ones_h, ones_v)
        plsc.subcore_barrier()

        def sup_body(i, carry):
            base = i * KDEPTH
            for t in range(KDEPTH):
                pltpu.async_copy(table.at[src_v.at[base + t]], bufs[t], sem_g)
            for t in range(KDEPTH):
                pltpu.make_async_copy(table.at[pl.ds(0, CHUNK)],
                                      bufs[t], sem_g).wait()
            for t in range(KDEPTH):
                pltpu.async_copy(bufs[t], accum.at[dst_v.at[base + t]],
                                 sem_s, add=True)
            if with_deg:
                for t in range(KDEPTH):
                    pltpu.async_copy(ones_v, dega.at[dst_v.at[base + t]],
                                     sem_d, add=True)
            for t in range(KDEPTH):
                pltpu.make_async_copy(bufs[t], accum.at[dst_v.at[0]],
                                      sem_s).wait()
            return carry

        lax.fori_loop(0, NSUP, sup_body, 0)
        if with_deg:
            def deg_drain(i, carry):
                pltpu.make_async_copy(ones_v, dega.at[dst_v.at[0]],
                                      sem_d).wait()
                return carry
            lax.fori_loop(0, CHUNKS_PER_W, deg_drain, 0)
        plsc.subcore_barrier()

        pltpu.sync_copy(accum.at[pl.ds(row0, ROWS_PER_S)],
                        psum.at[c, pl.ds(row0, ROWS_PER_S)])
        if with_deg:
            pltpu.sync_copy(dega.at[pl.ds(row0, ROWS_PER_S)], deg_v)
            pltpu.sync_copy(deg_v,
                            pdeg.at[pl.ds(c * N_PAD + row0, ROWS_PER_S)])

    out_type = [jax.ShapeDtypeStruct((NC, N_PAD, D), jnp.bfloat16)]
    scratch = [
        pltpu.VMEM_SHARED((N_PAD, D), jnp.bfloat16),
    ]
    if with_deg:
        out_type.append(jax.ShapeDtypeStruct((NC * N_PAD,), jnp.float32))
        scratch.append(pltpu.VMEM_SHARED((N_PAD,), jnp.float32))
    scratch += [
        pltpu.VMEM((CHUNKS_PER_W, CHUNK), jnp.int32),
        pltpu.VMEM((CHUNKS_PER_W, CHUNK), jnp.int32),
    ]
    if with_deg:
        scratch += [
            pltpu.VMEM((CHUNK,), jnp.float32),
            pltpu.VMEM((ROWS_PER_S,), jnp.float32),
        ]
    scratch += [pltpu.VMEM((CHUNK, D), jnp.bfloat16) for _ in range(KDEPTH)]
    scratch += [pltpu.SemaphoreType.DMA, pltpu.SemaphoreType.DMA]
    if with_deg:
        scratch.append(pltpu.SemaphoreType.DMA)

    return pl.kernel(
        body,
        mesh=plsc.VectorSubcoreMesh(core_axis_name="c", subcore_axis_name="s"),
        out_type=tuple(out_type) if with_deg else out_type[0],
        scratch_types=scratch,
        compiler_params=pltpu.CompilerParams(use_tc_tiling_on_sc=False),
    )


_segsum_deg = _make_segsum(True)
_segsum_nodeg = _make_segsum(False)


def _dense1_body(p0, p1, d0, d1, x, WlT, WrT, b, out):
    deg = jnp.maximum(d0[...] + d1[...], 1.0)
    agg = (p0[...].astype(jnp.float32) + p1[...].astype(jnp.float32)) / deg
    h = (jnp.dot(agg, WlT[...], preferred_element_type=jnp.float32)
         + jnp.dot(x[...], WrT[...], preferred_element_type=jnp.float32)
         + b[...])
    out[...] = jnp.maximum(h, 0.0)


def _dense2_body(p0, p1, d0, d1, x, WlT, WrT, b, WoT, bo, out):
    deg = jnp.maximum(d0[...] + d1[...], 1.0)
    agg = (p0[...].astype(jnp.float32) + p1[...].astype(jnp.float32)) / deg
    h = (jnp.dot(agg, WlT[...], preferred_element_type=jnp.float32)
         + jnp.dot(x[...], WrT[...], preferred_element_type=jnp.float32)
         + b[...])
    h = jnp.maximum(h, 0.0)
    out[...] = jnp.dot(h, WoT[...], preferred_element_type=jnp.float32) + bo[...]


def _row_specs():
    blk = lambda i: (i, 0)
    full = lambda i: (0, 0)
    return [
        pl.BlockSpec((ROW_BLOCK, D), blk),
        pl.BlockSpec((ROW_BLOCK, D), blk),
        pl.BlockSpec((ROW_BLOCK, 1), blk),
        pl.BlockSpec((ROW_BLOCK, 1), blk),
        pl.BlockSpec((ROW_BLOCK, D), blk),
        pl.BlockSpec((D, D), full),
        pl.BlockSpec((D, D), full),
        pl.BlockSpec((1, D), full),
    ]


def _dense1(p0, p1, d0, d1, x, WlT, WrT, b):
    grid = N_NODES // ROW_BLOCK
    return pl.pallas_call(
        _dense1_body,
        grid=(grid,),
        in_specs=_row_specs(),
        out_specs=pl.BlockSpec((ROW_BLOCK, D), lambda i: (i, 0)),
        out_shape=jax.ShapeDtypeStruct((N_NODES, D), jnp.float32),
    )(p0, p1, d0, d1, x, WlT, WrT, b)


def _dense2(p0, p1, d0, d1, x, WlT, WrT, b, WoT, bo):
    grid = N_NODES // ROW_BLOCK
    n_out = WoT.shape[1]
    in_specs = _row_specs() + [
        pl.BlockSpec((D, n_out), lambda i: (0, 0)),
        pl.BlockSpec((1, n_out), lambda i: (0, 0)),
    ]
    return pl.pallas_call(
        _dense2_body,
        grid=(grid,),
        in_specs=in_specs,
        out_specs=pl.BlockSpec((ROW_BLOCK, n_out), lambda i: (i, 0)),
        out_shape=jax.ShapeDtypeStruct((N_NODES, n_out), jnp.float32),
    )(p0, p1, d0, d1, x, WlT, WrT, b, WoT, bo)


def kernel(x, edge_index, W1l, b1, W1r, W2l, b2, W2r, Wlin, blin):
    ei = edge_index.astype(jnp.int32)
    pad = E_PAD - N_EDGES
    src = jnp.concatenate([ei[0], jnp.zeros((pad,), jnp.int32)])
    dst = jnp.concatenate([ei[1], jnp.full((pad,), N_NODES, jnp.int32)])
    src3 = src.reshape(NW, CHUNKS_PER_W, CHUNK)
    dst3 = dst.reshape(NW, CHUNKS_PER_W, CHUNK)
    zeros2 = jnp.zeros((N_PAD, D), jnp.bfloat16)
    zerosv = jnp.zeros((N_PAD,), jnp.float32)
    ones_h = jnp.ones((CHUNK,), jnp.float32)

    psum1, pdeg = _segsum_deg(x.astype(jnp.bfloat16), src3, dst3,
                              zeros2, zerosv, ones_h)
    pdeg = pdeg.reshape(NC, N_PAD)
    d0 = pdeg[0][:, None]
    d1 = pdeg[1][:, None]
    h1 = _dense1(psum1[0], psum1[1], d0, d1, x,
                 W1l.T, W1r.T, b1[None, :])

    psum2 = _segsum_nodeg(h1.astype(jnp.bfloat16), src3, dst3, zeros2)
    out = _dense2(psum2[0], psum2[1], d0, d1, h1,
                  W2l.T, W2r.T, b2[None, :], Wlin.T, blin[None, :])
    return out


# speedup vs baseline: 1.7961x; 1.0513x over previous
import jax
import jax.numpy as jnp
from jax import lax
from jax.experimental import pallas as pl
from jax.experimental.pallas import tpu as pltpu
from jax.experimental.pallas import tpu_sc as plsc

N_NODES = 10000
N_EDGES = 320000
D = 128

NC = 2
NS = 16
NW = NC * NS
CHUNK = 128
KDEPTH = 4
NBUF = 2 * KDEPTH
CHUNKS_PER_W = 80
NGRP = CHUNKS_PER_W // KDEPTH
EDGES_PER_W = CHUNKS_PER_W * CHUNK
E_PAD = EDGES_PER_W * NW
ROWS_PER_S = 632
N_PAD = ROWS_PER_S * NS

ROW_BLOCK = 1000


def _make_segsum(with_deg):
    def body(*refs):
        if with_deg:
            (table, src3, dst3, zeros2, zerosv, ones_h,
             psum, pdeg, accum, dega,
             src_v, dst_v, ones_v, deg_v) = refs[:14]
            bufs = refs[14:14 + NBUF]
            sem_ga, sem_gb, sem_sa, sem_sb, sem_d = refs[14 + NBUF:]
        else:
            (table, src3, dst3, zeros2,
             psum, accum,
             src_v, dst_v) = refs[:8]
            bufs = refs[8:8 + NBUF]
            sem_ga, sem_gb, sem_sa, sem_sb = refs[8 + NBUF:]

        c = lax.axis_index("c")
        s = lax.axis_index("s")
        wid = c * NS + s
        row0 = s * ROWS_PER_S

        pltpu.sync_copy(zeros2.at[pl.ds(row0, ROWS_PER_S)],
                        accum.at[pl.ds(row0, ROWS_PER_S)])
        pltpu.sync_copy(src3.at[wid], src_v)
        pltpu.sync_copy(dst3.at[wid], dst_v)
        if with_deg:
            pltpu.sync_copy(zerosv.at[pl.ds(row0, ROWS_PER_S)], deg_v)
            pltpu.sync_copy(deg_v, dega.at[pl.ds(row0, ROWS_PER_S)])
            pltpu.sync_copy(ones_h, ones_v)
        plsc.subcore_barrier()

        set_a, set_b = bufs[:KDEPTH], bufs[KDEPTH:]

        def fire_g(g, bset, sem):
            base = lax.rem(g, NGRP) * KDEPTH
            for t in range(KDEPTH):
                pltpu.async_copy(table.at[src_v.at[base + t]], bset[t], sem)

        def drain_g(bset, sem):
            for t in range(KDEPTH):
                pltpu.make_async_copy(table.at[pl.ds(0, CHUNK)],
                                      bset[t], sem).wait()

        def fire_s(g, bset, sem):
            base = g * KDEPTH
            for t in range(KDEPTH):
                pltpu.async_copy(bset[t], accum.at[dst_v.at[base + t]],
                                 sem, add=True)
            if with_deg:
                for t in range(KDEPTH):
                    pltpu.async_copy(ones_v, dega.at[dst_v.at[base + t]],
                                     sem_d, add=True)

        def drain_s(bset, sem):
            for t in range(KDEPTH):
                pltpu.make_async_copy(bset[t], accum.at[dst_v.at[0]],
                                      sem).wait()

        fire_g(0, set_a, sem_ga)
        drain_g(set_a, sem_ga)
        fire_s(0, set_a, sem_sa)
        fire_g(1, set_b, sem_gb)
        drain_g(set_b, sem_gb)
        fire_s(1, set_b, sem_sb)
        drain_s(set_a, sem_sa)
        fire_g(2, set_a, sem_ga)

        def pair_body(i, carry):
            g = 2 * i
            drain_g(set_a, sem_ga)
            fire_s(g, set_a, sem_sa)
            drain_s(set_b, sem_sb)
            fire_g(g + 1, set_b, sem_gb)
            drain_g(set_b, sem_gb)
            fire_s(g + 1, set_b, sem_sb)
            drain_s(set_a, sem_sa)
            fire_g(g + 2, set_a, sem_ga)
            return carry

        lax.fori_loop(1, NGRP // 2, pair_body, 0)
        drain_g(set_a, sem_ga)
        drain_s(set_b, sem_sb)
        if with_deg:
            def deg_drain(i, carry):
                pltpu.make_async_copy(ones_v, dega.at[dst_v.at[0]],
                                      sem_d).wait()
                return carry
            lax.fori_loop(0, CHUNKS_PER_W, deg_drain, 0)
        plsc.subcore_barrier()

        pltpu.sync_copy(accum.at[pl.ds(row0, ROWS_PER_S)],
                        psum.at[c, pl.ds(row0, ROWS_PER_S)])
        if with_deg:
            pltpu.sync_copy(dega.at[pl.ds(row0, ROWS_PER_S)], deg_v)
            pltpu.sync_copy(deg_v,
                            pdeg.at[pl.ds(c * N_PAD + row0, ROWS_PER_S)])

    out_type = [jax.ShapeDtypeStruct((NC, N_PAD, D), jnp.bfloat16)]
    scratch = [
        pltpu.VMEM_SHARED((N_PAD, D), jnp.bfloat16),
    ]
    if with_deg:
        out_type.append(jax.ShapeDtypeStruct((NC * N_PAD,), jnp.float32))
        scratch.append(pltpu.VMEM_SHARED((N_PAD,), jnp.float32))
    scratch += [
        pltpu.VMEM((CHUNKS_PER_W, CHUNK), jnp.int32),
        pltpu.VMEM((CHUNKS_PER_W, CHUNK), jnp.int32),
    ]
    if with_deg:
        scratch += [
            pltpu.VMEM((CHUNK,), jnp.float32),
            pltpu.VMEM((ROWS_PER_S,), jnp.float32),
        ]
    scratch += [pltpu.VMEM((CHUNK, D), jnp.bfloat16) for _ in range(NBUF)]
    scratch += [pltpu.SemaphoreType.DMA] * 4
    if with_deg:
        scratch.append(pltpu.SemaphoreType.DMA)

    return pl.kernel(
        body,
        mesh=plsc.VectorSubcoreMesh(core_axis_name="c", subcore_axis_name="s"),
        out_type=tuple(out_type) if with_deg else out_type[0],
        scratch_types=scratch,
        compiler_params=pltpu.CompilerParams(use_tc_tiling_on_sc=False),
    )


_segsum_deg = _make_segsum(True)
_segsum_nodeg = _make_segsum(False)


def _dense1_body(p0, p1, d0, d1, x, WlT, WrT, b, out):
    deg = jnp.maximum(d0[...] + d1[...], 1.0)
    agg = (p0[...].astype(jnp.float32) + p1[...].astype(jnp.float32)) / deg
    h = (jnp.dot(agg, WlT[...], preferred_element_type=jnp.float32)
         + jnp.dot(x[...], WrT[...], preferred_element_type=jnp.float32)
         + b[...])
    out[...] = jnp.maximum(h, 0.0)


def _dense2_body(p0, p1, d0, d1, x, WlT, WrT, b, WoT, bo, out):
    deg = jnp.maximum(d0[...] + d1[...], 1.0)
    agg = (p0[...].astype(jnp.float32) + p1[...].astype(jnp.float32)) / deg
    h = (jnp.dot(agg, WlT[...], preferred_element_type=jnp.float32)
         + jnp.dot(x[...], WrT[...], preferred_element_type=jnp.float32)
         + b[...])
    h = jnp.maximum(h, 0.0)
    out[...] = jnp.dot(h, WoT[...], preferred_element_type=jnp.float32) + bo[...]


def _row_specs():
    blk = lambda i: (i, 0)
    full = lambda i: (0, 0)
    return [
        pl.BlockSpec((ROW_BLOCK, D), blk),
        pl.BlockSpec((ROW_BLOCK, D), blk),
        pl.BlockSpec((ROW_BLOCK, 1), blk),
        pl.BlockSpec((ROW_BLOCK, 1), blk),
        pl.BlockSpec((ROW_BLOCK, D), blk),
        pl.BlockSpec((D, D), full),
        pl.BlockSpec((D, D), full),
        pl.BlockSpec((1, D), full),
    ]


def _dense1(p0, p1, d0, d1, x, WlT, WrT, b):
    grid = N_NODES // ROW_BLOCK
    return pl.pallas_call(
        _dense1_body,
        grid=(grid,),
        in_specs=_row_specs(),
        out_specs=pl.BlockSpec((ROW_BLOCK, D), lambda i: (i, 0)),
        out_shape=jax.ShapeDtypeStruct((N_NODES, D), jnp.float32),
    )(p0, p1, d0, d1, x, WlT, WrT, b)


def _dense2(p0, p1, d0, d1, x, WlT, WrT, b, WoT, bo):
    grid = N_NODES // ROW_BLOCK
    n_out = WoT.shape[1]
    in_specs = _row_specs() + [
        pl.BlockSpec((D, n_out), lambda i: (0, 0)),
        pl.BlockSpec((1, n_out), lambda i: (0, 0)),
    ]
    return pl.pallas_call(
        _dense2_body,
        grid=(grid,),
        in_specs=in_specs,
        out_specs=pl.BlockSpec((ROW_BLOCK, n_out), lambda i: (i, 0)),
        out_shape=jax.ShapeDtypeStruct((N_NODES, n_out), jnp.float32),
    )(p0, p1, d0, d1, x, WlT, WrT, b, WoT, bo)


def kernel(x, edge_index, W1l, b1, W1r, W2l, b2, W2r, Wlin, blin):
    ei = edge_index.astype(jnp.int32)
    pad = E_PAD - N_EDGES
    src = jnp.concatenate([ei[0], jnp.zeros((pad,), jnp.int32)])
    dst = jnp.concatenate([ei[1], jnp.full((pad,), N_NODES, jnp.int32)])
    src3 = src.reshape(NW, CHUNKS_PER_W, CHUNK)
    dst3 = dst.reshape(NW, CHUNKS_PER_W, CHUNK)
    zeros2 = jnp.zeros((N_PAD, D), jnp.bfloat16)
    zerosv = jnp.zeros((N_PAD,), jnp.float32)
    ones_h = jnp.ones((CHUNK,), jnp.float32)

    psum1, pdeg = _segsum_deg(x.astype(jnp.bfloat16), src3, dst3,
                              zeros2, zerosv, ones_h)
    pdeg = pdeg.reshape(NC, N_PAD)
    d0 = pdeg[0][:, None]
    d1 = pdeg[1][:, None]
    h1 = _dense1(psum1[0], psum1[1], d0, d1, x,
                 W1l.T, W1r.T, b1[None, :])

    psum2 = _segsum_nodeg(h1.astype(jnp.bfloat16), src3, dst3, zeros2)
    out = _dense2(psum2[0], psum2[1], d0, d1, h1,
                  W2l.T, W2r.T, b2[None, :], Wlin.T, blin[None, :])
    return out
